# baseline (speedup 1.0000x reference)
import jax
import jax.numpy as jnp
from jax import lax
from jax.experimental import pallas as pl
from jax.experimental.pallas import tpu as pltpu, tpu_sc as plsc

DIM = 3
N_NODES = 100000
NBR = 32
LANES = 16
NUM_WORKERS = 32
CHUNK = 256
ALIGNED = (N_NODES // CHUNK) * CHUNK
TAIL = N_NODES - ALIGNED
TC_BLK = 16384
SPLIT = 75264
SC_FULL = (ALIGNED - SPLIT) // CHUNK
SC_N = N_NODES - SPLIT
assert SC_FULL == 3 * NUM_WORKERS and SPLIT % CHUNK == 0
EPS = 1e-8
LAM = 1e-6


def _accum_solve(loads, store, g):
    axx = ayy = azz = axy = axz = ayz = None
    bx = by = bz = None
    for n in range(NBR):
        x, y, z, d = loads(n)
        w = 1.0 / (x * x + y * y + z * z + EPS)
        wx = w * x
        wy = w * y
        wz = w * z
        if n == 0:
            axx, ayy, azz = wx * x, wy * y, wz * z
            axy, axz, ayz = wx * y, wx * z, wy * z
            bx, by, bz = wx * d, wy * d, wz * d
        else:
            axx += wx * x
            ayy += wy * y
            azz += wz * z
            axy += wx * y
            axz += wx * z
            ayz += wy * z
            bx += wx * d
            by += wy * d
            bz += wz * d
    axx += LAM
    ayy += LAM
    azz += LAM
    c00 = ayy * azz - ayz * ayz
    c01 = axz * ayz - axy * azz
    c02 = axy * ayz - axz * ayy
    c11 = axx * azz - axz * axz
    c12 = axy * axz - axx * ayz
    c22 = axx * ayy - axy * axy
    inv_det = 1.0 / (axx * c00 + axy * c01 + axz * c02)
    store((c00 * bx + c01 * by + c02 * bz) * inv_det,
          (c01 * bx + c11 * by + c12 * bz) * inv_det,
          (c02 * bx + c12 * by + c22 * bz) * inv_det)


def _body(dvt_hbm, dut_hbm, dvf_hbm, duf_hbm, ox_hbm, oy_hbm, oz_hbm,
          dv_v, du_v, ox_v, oy_v, oz_v,
          dv_t, du_t, ox_t, oy_t, oz_t,
          sem_a, sem_b, sem_oa, sem_ob):
    wid = lax.axis_index("s") * 2 + lax.axis_index("c")
    sems = (sem_a, sem_b)
    osems = (sem_oa, sem_ob)
    outs = ((ox_v, ox_hbm), (oy_v, oy_hbm), (oz_v, oz_hbm))

    def chunk_start(m):
        return pl.multiple_of((wid + m * NUM_WORKERS) * CHUNK, 128)

    def start_in(m, slot):
        s = pl.multiple_of(SPLIT + chunk_start(m), 128)
        for i in range(DIM):
            pltpu.async_copy(dvt_hbm.at[i, :, pl.ds(s, CHUNK)],
                             dv_v.at[slot, i], sems[slot])
        pltpu.async_copy(dut_hbm.at[:, pl.ds(s, CHUNK)], du_v.at[slot], sems[slot])

    def wait_in(slot):
        for i in range(DIM):
            pltpu.make_async_copy(dvt_hbm.at[i, :, pl.ds(0, CHUNK)],
                                  dv_v.at[slot, i], sems[slot]).wait()
        pltpu.make_async_copy(dut_hbm.at[:, pl.ds(0, CHUNK)],
                              du_v.at[slot], sems[slot]).wait()

    def start_out(m, slot):
        s = chunk_start(m)
        for buf, hbm in outs:
            pltpu.async_copy(buf.at[slot], hbm.at[pl.ds(s, CHUNK)], osems[slot])

    def wait_out(slot):
        for buf, hbm in outs:
            pltpu.make_async_copy(buf.at[slot], hbm.at[pl.ds(0, CHUNK)],
                                  osems[slot]).wait()

    def compute(slot):
        @plsc.parallel_loop(0, CHUNK // LANES, unroll=2)
        def group(g):
            sl = pl.ds(g * LANES, LANES)

            def loads(n):
                return (dv_v[slot, 0, n, sl], dv_v[slot, 1, n, sl],
                        dv_v[slot, 2, n, sl], du_v[slot, n, sl])

            def store(gx, gy, gz):
                ox_v[slot, sl] = gx
                oy_v[slot, sl] = gy
                oz_v[slot, sl] = gz

            _accum_solve(loads, store, g)

    def process_tail_group():
        g = wid
        pltpu.sync_copy(dvf_hbm.at[pl.ds(g * (LANES * 3 * NBR), LANES * 3 * NBR)],
                        dv_t)
        pltpu.sync_copy(duf_hbm.at[pl.ds(g * (LANES * NBR), LANES * NBR)], du_t)
        iota = jnp.arange(LANES, dtype=jnp.int32)
        dvb = iota * (3 * NBR)
        dub = iota * NBR

        def loads(n):
            ix = dvb + 3 * n
            return (plsc.load_gather(dv_t, [ix]),
                    plsc.load_gather(dv_t, [ix + 1]),
                    plsc.load_gather(dv_t, [ix + 2]),
                    plsc.load_gather(du_t, [dub + n]))

        def store(gx, gy, gz):
            ox_t[...] = gx
            oy_t[...] = gy
            oz_t[...] = gz

        _accum_solve(loads, store, 0)
        base = SC_FULL * CHUNK + g * LANES
        pltpu.sync_copy(ox_t, ox_hbm.at[pl.ds(base, LANES)])
        pltpu.sync_copy(oy_t, oy_hbm.at[pl.ds(base, LANES)])
        pltpu.sync_copy(oz_t, oz_hbm.at[pl.ds(base, LANES)])

    start_in(0, 0)
    start_in(1, 1)
    wait_in(0)
    compute(0)
    start_out(0, 0)
    start_in(2, 0)
    wait_in(1)
    compute(1)
    start_out(1, 1)
    wait_in(0)
    wait_out(0)
    compute(0)
    start_out(2, 0)

    @pl.when(wid < TAIL // LANES)
    def _tail():
        process_tail_group()

    wait_out(0)
    wait_out(1)


def _sc_grads(dvt, dut, dvf, duf):
    f32 = jnp.float32
    run = pl.kernel(
        _body,
        out_type=(
            jax.ShapeDtypeStruct((SC_N,), f32),
            jax.ShapeDtypeStruct((SC_N,), f32),
            jax.ShapeDtypeStruct((SC_N,), f32),
        ),
        mesh=plsc.VectorSubcoreMesh(core_axis_name="c", subcore_axis_name="s"),
        compiler_params=pltpu.CompilerParams(needs_layout_passes=False),
        scratch_types=[
            pltpu.VMEM((2, DIM, NBR, CHUNK), f32),
            pltpu.VMEM((2, NBR, CHUNK), f32),
            pltpu.VMEM((2, CHUNK), f32),
            pltpu.VMEM((2, CHUNK), f32),
            pltpu.VMEM((2, CHUNK), f32),
            pltpu.VMEM((LANES * 3 * NBR,), f32),
            pltpu.VMEM((LANES * NBR,), f32),
            pltpu.VMEM((LANES,), f32),
            pltpu.VMEM((LANES,), f32),
            pltpu.VMEM((LANES,), f32),
            pltpu.SemaphoreType.DMA,
            pltpu.SemaphoreType.DMA,
            pltpu.SemaphoreType.DMA,
            pltpu.SemaphoreType.DMA,
        ],
    )
    return run(dvt, dut, dvf, duf)


def _tc_body(dvt_ref, dut_ref, ox_ref, oy_ref, oz_ref):
    def loads(n):
        return (dvt_ref[0, n, :], dvt_ref[1, n, :],
                dvt_ref[2, n, :], dut_ref[n, :])

    def store(gx, gy, gz):
        ox_ref[:] = gx
        oy_ref[:] = gy
        oz_ref[:] = gz

    _accum_solve(loads, store, 0)


def _tc_grads(dvt, dut):
    f32 = jnp.float32
    return pl.pallas_call(
        _tc_body,
        grid=(-(-SPLIT // TC_BLK),),
        in_specs=[
            pl.BlockSpec((DIM, NBR, TC_BLK), lambda i: (0, 0, i)),
            pl.BlockSpec((NBR, TC_BLK), lambda i: (0, i)),
        ],
        out_specs=[pl.BlockSpec((TC_BLK,), lambda i: (i,))] * 3,
        out_shape=(
            jax.ShapeDtypeStruct((SPLIT,), f32),
            jax.ShapeDtypeStruct((SPLIT,), f32),
            jax.ShapeDtypeStruct((SPLIT,), f32),
        ),
    )(dvt, dut)


@jax.jit
def _ls_grads(dvt, dut, dvf, duf):
    sx, sy, sz = _sc_grads(dvt, dut, dvf, duf)
    tx, ty, tz = _tc_grads(dvt, dut)
    return (jnp.concatenate([tx, sx]), jnp.concatenate([ty, sy]),
            jnp.concatenate([tz, sz]))


def kernel(coords, connectivity_tensor, y, du, dv):
    del coords, connectivity_tensor, y
    dvt = jnp.transpose(dv, (2, 1, 0))
    dut = jnp.transpose(du[:, :, 0], (1, 0))
    dvf = dv[ALIGNED:].reshape(-1)
    duf = du[ALIGNED:].reshape(-1)
    gx, gy, gz = _ls_grads(dvt, dut, dvf, duf)
    return (gx[:, None], gy[:, None], gz[:, None])


# speedup vs baseline: 1.0196x; 1.0196x over previous
import jax
import jax.numpy as jnp
from jax import lax
from jax.experimental import pallas as pl
from jax.experimental.pallas import tpu as pltpu, tpu_sc as plsc

DIM = 3
N_NODES = 100000
NBR = 32
LANES = 16
NUM_WORKERS = 32
CHUNK = 256
ALIGNED = (N_NODES // CHUNK) * CHUNK
TAIL = N_NODES - ALIGNED
TC_BLK = 4096
SPLIT = 75264
SC_FULL = (ALIGNED - SPLIT) // CHUNK
SC_N = N_NODES - SPLIT
assert SC_FULL == 3 * NUM_WORKERS and SPLIT % CHUNK == 0
EPS = 1e-8
LAM = 1e-6


def _accum_solve(loads, store, g):
    axx = ayy = azz = axy = axz = ayz = None
    bx = by = bz = None
    for n in range(NBR):
        x, y, z, d = loads(n)
        w = 1.0 / (x * x + y * y + z * z + EPS)
        wx = w * x
        wy = w * y
        wz = w * z
        if n == 0:
            axx, ayy, azz = wx * x, wy * y, wz * z
            axy, axz, ayz = wx * y, wx * z, wy * z
            bx, by, bz = wx * d, wy * d, wz * d
        else:
            axx += wx * x
            ayy += wy * y
            azz += wz * z
            axy += wx * y
            axz += wx * z
            ayz += wy * z
            bx += wx * d
            by += wy * d
            bz += wz * d
    axx += LAM
    ayy += LAM
    azz += LAM
    c00 = ayy * azz - ayz * ayz
    c01 = axz * ayz - axy * azz
    c02 = axy * ayz - axz * ayy
    c11 = axx * azz - axz * axz
    c12 = axy * axz - axx * ayz
    c22 = axx * ayy - axy * axy
    inv_det = 1.0 / (axx * c00 + axy * c01 + axz * c02)
    store((c00 * bx + c01 * by + c02 * bz) * inv_det,
          (c01 * bx + c11 * by + c12 * bz) * inv_det,
          (c02 * bx + c12 * by + c22 * bz) * inv_det)


def _body(dvt_hbm, dut_hbm, dvf_hbm, duf_hbm, ox_hbm, oy_hbm, oz_hbm,
          dv_v, du_v, ox_v, oy_v, oz_v,
          dv_t, du_t, ox_t, oy_t, oz_t,
          sem_a, sem_b, sem_oa, sem_ob):
    wid = lax.axis_index("s") * 2 + lax.axis_index("c")
    sems = (sem_a, sem_b)
    osems = (sem_oa, sem_ob)
    outs = ((ox_v, ox_hbm), (oy_v, oy_hbm), (oz_v, oz_hbm))

    def chunk_start(m):
        return pl.multiple_of((wid + m * NUM_WORKERS) * CHUNK, 128)

    def start_in(m, slot):
        s = pl.multiple_of(SPLIT + chunk_start(m), 128)
        for i in range(DIM):
            pltpu.async_copy(dvt_hbm.at[i, :, pl.ds(s, CHUNK)],
                             dv_v.at[slot, i], sems[slot])
        pltpu.async_copy(dut_hbm.at[:, pl.ds(s, CHUNK)], du_v.at[slot], sems[slot])

    def wait_in(slot):
        for i in range(DIM):
            pltpu.make_async_copy(dvt_hbm.at[i, :, pl.ds(0, CHUNK)],
                                  dv_v.at[slot, i], sems[slot]).wait()
        pltpu.make_async_copy(dut_hbm.at[:, pl.ds(0, CHUNK)],
                              du_v.at[slot], sems[slot]).wait()

    def start_out(m, slot):
        s = chunk_start(m)
        for buf, hbm in outs:
            pltpu.async_copy(buf.at[slot], hbm.at[pl.ds(s, CHUNK)], osems[slot])

    def wait_out(slot):
        for buf, hbm in outs:
            pltpu.make_async_copy(buf.at[slot], hbm.at[pl.ds(0, CHUNK)],
                                  osems[slot]).wait()

    def compute(slot):
        @plsc.parallel_loop(0, CHUNK // LANES, unroll=2)
        def group(g):
            sl = pl.ds(g * LANES, LANES)

            def loads(n):
                return (dv_v[slot, 0, n, sl], dv_v[slot, 1, n, sl],
                        dv_v[slot, 2, n, sl], du_v[slot, n, sl])

            def store(gx, gy, gz):
                ox_v[slot, sl] = gx
                oy_v[slot, sl] = gy
                oz_v[slot, sl] = gz

            _accum_solve(loads, store, g)

    def process_tail_group():
        g = wid
        pltpu.sync_copy(dvf_hbm.at[pl.ds(g * (LANES * 3 * NBR), LANES * 3 * NBR)],
                        dv_t)
        pltpu.sync_copy(duf_hbm.at[pl.ds(g * (LANES * NBR), LANES * NBR)], du_t)
        iota = jnp.arange(LANES, dtype=jnp.int32)
        dvb = iota * (3 * NBR)
        dub = iota * NBR

        def loads(n):
            ix = dvb + 3 * n
            return (plsc.load_gather(dv_t, [ix]),
                    plsc.load_gather(dv_t, [ix + 1]),
                    plsc.load_gather(dv_t, [ix + 2]),
                    plsc.load_gather(du_t, [dub + n]))

        def store(gx, gy, gz):
            ox_t[...] = gx
            oy_t[...] = gy
            oz_t[...] = gz

        _accum_solve(loads, store, 0)
        base = SC_FULL * CHUNK + g * LANES
        pltpu.sync_copy(ox_t, ox_hbm.at[pl.ds(base, LANES)])
        pltpu.sync_copy(oy_t, oy_hbm.at[pl.ds(base, LANES)])
        pltpu.sync_copy(oz_t, oz_hbm.at[pl.ds(base, LANES)])

    start_in(0, 0)
    start_in(1, 1)
    wait_in(0)
    compute(0)
    start_out(0, 0)
    start_in(2, 0)
    wait_in(1)
    compute(1)
    start_out(1, 1)
    wait_in(0)
    wait_out(0)
    compute(0)
    start_out(2, 0)

    @pl.when(wid < TAIL // LANES)
    def _tail():
        process_tail_group()

    wait_out(0)
    wait_out(1)


def _sc_grads(dvt, dut, dvf, duf):
    f32 = jnp.float32
    run = pl.kernel(
        _body,
        out_type=(
            jax.ShapeDtypeStruct((SC_N,), f32),
            jax.ShapeDtypeStruct((SC_N,), f32),
            jax.ShapeDtypeStruct((SC_N,), f32),
        ),
        mesh=plsc.VectorSubcoreMesh(core_axis_name="c", subcore_axis_name="s"),
        compiler_params=pltpu.CompilerParams(needs_layout_passes=False),
        scratch_types=[
            pltpu.VMEM((2, DIM, NBR, CHUNK), f32),
            pltpu.VMEM((2, NBR, CHUNK), f32),
            pltpu.VMEM((2, CHUNK), f32),
            pltpu.VMEM((2, CHUNK), f32),
            pltpu.VMEM((2, CHUNK), f32),
            pltpu.VMEM((LANES * 3 * NBR,), f32),
            pltpu.VMEM((LANES * NBR,), f32),
            pltpu.VMEM((LANES,), f32),
            pltpu.VMEM((LANES,), f32),
            pltpu.VMEM((LANES,), f32),
            pltpu.SemaphoreType.DMA,
            pltpu.SemaphoreType.DMA,
            pltpu.SemaphoreType.DMA,
            pltpu.SemaphoreType.DMA,
        ],
    )
    return run(dvt, dut, dvf, duf)


def _tc_body(dvt_ref, dut_ref, ox_ref, oy_ref, oz_ref):
    def loads(n):
        return (dvt_ref[0, n, :], dvt_ref[1, n, :],
                dvt_ref[2, n, :], dut_ref[n, :])

    def store(gx, gy, gz):
        ox_ref[:] = gx
        oy_ref[:] = gy
        oz_ref[:] = gz

    _accum_solve(loads, store, 0)


def _tc_grads(dvt, dut):
    f32 = jnp.float32
    return pl.pallas_call(
        _tc_body,
        grid=(-(-SPLIT // TC_BLK),),
        in_specs=[
            pl.BlockSpec((DIM, NBR, TC_BLK), lambda i: (0, 0, i)),
            pl.BlockSpec((NBR, TC_BLK), lambda i: (0, i)),
        ],
        out_specs=[pl.BlockSpec((TC_BLK,), lambda i: (i,))] * 3,
        out_shape=(
            jax.ShapeDtypeStruct((SPLIT,), f32),
            jax.ShapeDtypeStruct((SPLIT,), f32),
            jax.ShapeDtypeStruct((SPLIT,), f32),
        ),
    )(dvt, dut)


@jax.jit
def _ls_grads(dvt, dut, dvf, duf):
    sx, sy, sz = _sc_grads(dvt, dut, dvf, duf)
    tx, ty, tz = _tc_grads(dvt, dut)
    return (jnp.concatenate([tx, sx]), jnp.concatenate([ty, sy]),
            jnp.concatenate([tz, sz]))


def kernel(coords, connectivity_tensor, y, du, dv):
    del coords, connectivity_tensor, y
    dvt = jnp.transpose(dv, (2, 1, 0))
    dut = jnp.transpose(du[:, :, 0], (1, 0))
    dvf = dv[ALIGNED:].reshape(-1)
    duf = du[ALIGNED:].reshape(-1)
    gx, gy, gz = _ls_grads(dvt, dut, dvf, duf)
    return (gx[:, None], gy[:, None], gz[:, None])


# speedup vs baseline: 1.0238x; 1.0041x over previous
import jax
import jax.numpy as jnp
from jax import lax
from jax.experimental import pallas as pl
from jax.experimental.pallas import tpu as pltpu, tpu_sc as plsc

DIM = 3
N_NODES = 100000
NBR = 32
LANES = 16
NUM_WORKERS = 32
CHUNK = 256
ALIGNED = (N_NODES // CHUNK) * CHUNK
TAIL = N_NODES - ALIGNED
TC_BLK = 4096
SPLIT = 75264
SC_FULL = (ALIGNED - SPLIT) // CHUNK
SC_N = N_NODES - SPLIT
assert SC_FULL == 3 * NUM_WORKERS and SPLIT % CHUNK == 0
EPS = 1e-8
LAM = 1e-6


def _accum_solve(loads, store):
    axx = ayy = azz = axy = axz = ayz = None
    bx = by = bz = None
    for n in range(NBR):
        x, y, z, d = loads(n)
        w = 1.0 / (x * x + y * y + z * z + EPS)
        wx = w * x
        wy = w * y
        wz = w * z
        if n == 0:
            axx, ayy, azz = wx * x, wy * y, wz * z
            axy, axz, ayz = wx * y, wx * z, wy * z
            bx, by, bz = wx * d, wy * d, wz * d
        else:
            axx += wx * x
            ayy += wy * y
            azz += wz * z
            axy += wx * y
            axz += wx * z
            ayz += wy * z
            bx += wx * d
            by += wy * d
            bz += wz * d
    axx += LAM
    ayy += LAM
    azz += LAM
    c00 = ayy * azz - ayz * ayz
    c01 = axz * ayz - axy * azz
    c02 = axy * ayz - axz * ayy
    c11 = axx * azz - axz * axz
    c12 = axy * axz - axx * ayz
    c22 = axx * ayy - axy * axy
    inv_det = 1.0 / (axx * c00 + axy * c01 + axz * c02)
    store((c00 * bx + c01 * by + c02 * bz) * inv_det,
          (c01 * bx + c11 * by + c12 * bz) * inv_det,
          (c02 * bx + c12 * by + c22 * bz) * inv_det)


def _body(dvt_hbm, dut_hbm, dvf_hbm, duf_hbm, ox_hbm, oy_hbm, oz_hbm,
          dv_v, du_v, ox_v, oy_v, oz_v,
          dv_t, du_t, ox_t, oy_t, oz_t,
          sem_a, sem_b, sem_oa, sem_ob):
    wid = lax.axis_index("s") * 2 + lax.axis_index("c")
    sems = (sem_a, sem_b)
    osems = (sem_oa, sem_ob)
    outs = ((ox_v, ox_hbm), (oy_v, oy_hbm), (oz_v, oz_hbm))

    def chunk_start(m):
        return pl.multiple_of((wid + m * NUM_WORKERS) * CHUNK, 128)

    def start_in(m, slot):
        s = pl.multiple_of(SPLIT + chunk_start(m), 128)
        for i in range(DIM):
            pltpu.async_copy(dvt_hbm.at[i, :, pl.ds(s, CHUNK)],
                             dv_v.at[slot, i], sems[slot])
        pltpu.async_copy(dut_hbm.at[:, pl.ds(s, CHUNK)], du_v.at[slot], sems[slot])

    def wait_in(slot):
        for i in range(DIM):
            pltpu.make_async_copy(dvt_hbm.at[i, :, pl.ds(0, CHUNK)],
                                  dv_v.at[slot, i], sems[slot]).wait()
        pltpu.make_async_copy(dut_hbm.at[:, pl.ds(0, CHUNK)],
                              du_v.at[slot], sems[slot]).wait()

    def start_out(m, slot):
        s = chunk_start(m)
        for buf, hbm in outs:
            pltpu.async_copy(buf.at[slot], hbm.at[pl.ds(s, CHUNK)], osems[slot])

    def wait_out(slot):
        for buf, hbm in outs:
            pltpu.make_async_copy(buf.at[slot], hbm.at[pl.ds(0, CHUNK)],
                                  osems[slot]).wait()

    def compute(slot):
        @plsc.parallel_loop(0, CHUNK // LANES, unroll=2)
        def group(g):
            sl = pl.ds(g * LANES, LANES)

            def loads(n):
                return (dv_v[slot, 0, n, sl], dv_v[slot, 1, n, sl],
                        dv_v[slot, 2, n, sl], du_v[slot, n, sl])

            def store(gx, gy, gz):
                ox_v[slot, sl] = gx
                oy_v[slot, sl] = gy
                oz_v[slot, sl] = gz

            _accum_solve(loads, store)

    def process_tail_group():
        g = wid
        pltpu.sync_copy(dvf_hbm.at[pl.ds(g * (LANES * 3 * NBR), LANES * 3 * NBR)],
                        dv_t)
        pltpu.sync_copy(duf_hbm.at[pl.ds(g * (LANES * NBR), LANES * NBR)], du_t)
        iota = jnp.arange(LANES, dtype=jnp.int32)
        dvb = iota * (3 * NBR)
        dub = iota * NBR

        def loads(n):
            ix = dvb + 3 * n
            return (plsc.load_gather(dv_t, [ix]),
                    plsc.load_gather(dv_t, [ix + 1]),
                    plsc.load_gather(dv_t, [ix + 2]),
                    plsc.load_gather(du_t, [dub + n]))

        def store(gx, gy, gz):
            ox_t[...] = gx
            oy_t[...] = gy
            oz_t[...] = gz

        _accum_solve(loads, store)
        base = SC_FULL * CHUNK + g * LANES
        pltpu.sync_copy(ox_t, ox_hbm.at[pl.ds(base, LANES)])
        pltpu.sync_copy(oy_t, oy_hbm.at[pl.ds(base, LANES)])
        pltpu.sync_copy(oz_t, oz_hbm.at[pl.ds(base, LANES)])

    start_in(0, 0)
    start_in(1, 1)
    wait_in(0)
    compute(0)
    start_out(0, 0)
    start_in(2, 0)
    wait_in(1)
    compute(1)
    start_out(1, 1)
    wait_in(0)
    wait_out(0)
    compute(0)
    start_out(2, 0)

    @pl.when(wid < TAIL // LANES)
    def _tail():
        process_tail_group()

    wait_out(0)
    wait_out(1)


def _sc_grads(dvt, dut, dvf, duf):
    f32 = jnp.float32
    run = pl.kernel(
        _body,
        out_type=(
            jax.ShapeDtypeStruct((SC_N,), f32),
            jax.ShapeDtypeStruct((SC_N,), f32),
            jax.ShapeDtypeStruct((SC_N,), f32),
        ),
        mesh=plsc.VectorSubcoreMesh(core_axis_name="c", subcore_axis_name="s"),
        compiler_params=pltpu.CompilerParams(needs_layout_passes=False),
        scratch_types=[
            pltpu.VMEM((2, DIM, NBR, CHUNK), f32),
            pltpu.VMEM((2, NBR, CHUNK), f32),
            pltpu.VMEM((2, CHUNK), f32),
            pltpu.VMEM((2, CHUNK), f32),
            pltpu.VMEM((2, CHUNK), f32),
            pltpu.VMEM((LANES * 3 * NBR,), f32),
            pltpu.VMEM((LANES * NBR,), f32),
            pltpu.VMEM((LANES,), f32),
            pltpu.VMEM((LANES,), f32),
            pltpu.VMEM((LANES,), f32),
            pltpu.SemaphoreType.DMA,
            pltpu.SemaphoreType.DMA,
            pltpu.SemaphoreType.DMA,
            pltpu.SemaphoreType.DMA,
        ],
    )
    return run(dvt, dut, dvf, duf)


def _tc_body(dvt_ref, dut_ref, ox_ref, oy_ref, oz_ref):
    def loads(n):
        return (dvt_ref[0, n, :], dvt_ref[1, n, :],
                dvt_ref[2, n, :], dut_ref[n, :])

    def store(gx, gy, gz):
        ox_ref[:] = gx
        oy_ref[:] = gy
        oz_ref[:] = gz

    _accum_solve(loads, store)


def _tc_grads(dvt, dut):
    f32 = jnp.float32
    return pl.pallas_call(
        _tc_body,
        grid=(-(-SPLIT // TC_BLK),),
        in_specs=[
            pl.BlockSpec((DIM, NBR, TC_BLK), lambda i: (0, 0, i)),
            pl.BlockSpec((NBR, TC_BLK), lambda i: (0, i)),
        ],
        out_specs=[pl.BlockSpec((TC_BLK,), lambda i: (i,))] * 3,
        out_shape=(
            jax.ShapeDtypeStruct((SPLIT,), f32),
            jax.ShapeDtypeStruct((SPLIT,), f32),
            jax.ShapeDtypeStruct((SPLIT,), f32),
        ),
    )(dvt, dut)


@jax.jit
def _ls_grads(dvt, dut, dvf, duf):
    sx, sy, sz = _sc_grads(dvt, dut, dvf, duf)
    tx, ty, tz = _tc_grads(dvt, dut)
    return (jnp.concatenate([tx, sx]), jnp.concatenate([ty, sy]),
            jnp.concatenate([tz, sz]))


def kernel(coords, connectivity_tensor, y, du, dv):
    del coords, connectivity_tensor, y
    dvt = jnp.transpose(dv, (2, 1, 0))
    dut = jnp.transpose(du[:, :, 0], (1, 0))
    dvf = dv[ALIGNED:].reshape(-1)
    duf = du[ALIGNED:].reshape(-1)
    gx, gy, gz = _ls_grads(dvt, dut, dvf, duf)
    return (gx[:, None], gy[:, None], gz[:, None])
